# Initial kernel scaffold; baseline (speedup 1.0000x reference)
import functools

import jax
import jax.numpy as jnp
from jax import lax
from jax.experimental import pallas as pl
from jax.experimental.pallas import tpu as pltpu
from jax.experimental.pallas import tpu_sc as plsc

MIN_POS = 0.0
MAX_POS = 1.0
N_POS = 100000
N_HEADS = 16

NC = 2
NS = 16
NW = NC * NS
L = 16

CHUNK = 2048
ROW = 128
ROWS = CHUNK // ROW


def _make_sc_kernel(n_total):
    per_w = n_total // NW
    chunks = per_w // CHUNK
    mesh = plsc.VectorSubcoreMesh(
        core_axis_name="c", subcore_axis_name="s", num_cores=NC, num_subcores=NS
    )

    @functools.partial(
        pl.kernel,
        out_type=(
            jax.ShapeDtypeStruct((n_total, N_HEADS), jnp.float32),
            jax.ShapeDtypeStruct((n_total // ROW, ROW), jnp.int32),
        ),
        mesh=mesh,
        scratch_types=[
            pltpu.VMEM((CHUNK,), jnp.float32),
            pltpu.VMEM((ROWS, ROW), jnp.int32),
            pltpu.VMEM((CHUNK, N_HEADS), jnp.float32),
            pltpu.SemaphoreType.DMA,
        ],
    )
    def body(coord_hbm, table_hbm, emb_hbm, idx_hbm, coord_v, idx_v, rows_v, sem):
        wid = lax.axis_index("s") * NC + lax.axis_index("c")
        wbase = wid * per_w
        scale = jnp.float32(N_POS / (MAX_POS - MIN_POS))

        def chunk_body(c, carry):
            base = wbase + c * CHUNK
            pltpu.sync_copy(coord_hbm.at[pl.ds(base, CHUNK)], coord_v)
            descs = []
            for j in range(ROWS):
                for u in range(ROW // L):
                    v = coord_v[pl.ds(j * ROW + u * L, L)]
                    pos = jnp.clip((v - MIN_POS) * scale, 0.0, float(N_POS))
                    idx_v[j, pl.ds(u * L, L)] = pos.astype(jnp.int32)
                descs.append(
                    pltpu.async_copy(
                        table_hbm.at[idx_v.at[j]],
                        rows_v.at[pl.ds(j * ROW, ROW)],
                        sem,
                    )
                )
            for d in descs:
                d.wait()
            pltpu.sync_copy(rows_v, emb_hbm.at[pl.ds(base, CHUNK)])
            pltpu.sync_copy(idx_v, idx_hbm.at[pl.ds(base // ROW, ROWS)])
            return carry

        lax.fori_loop(0, chunks, chunk_body, 0, unroll=False)

    return body


def kernel(coord, embeddings_table):
    m, k = coord.shape
    n_total = m * k
    coord_flat = coord.reshape(n_total)
    emb_flat, idx_2d = _make_sc_kernel(n_total)(coord_flat, embeddings_table)
    return emb_flat.reshape(m, k, N_HEADS), idx_2d.reshape(m, k)


# speedup vs baseline: 6.1164x; 6.1164x over previous
import functools

import jax
import jax.numpy as jnp
from jax import lax
from jax.experimental import pallas as pl
from jax.experimental.pallas import tpu as pltpu
from jax.experimental.pallas import tpu_sc as plsc

MIN_POS = 0.0
MAX_POS = 1.0
N_POS = 100000
N_HEADS = 16

NC = 2
NS = 16
NW = NC * NS
L = 16

CHUNK = 2048
ROW = 128
ROWS = CHUNK // ROW


def _make_sc_kernel(n_total):
    per_w = n_total // NW
    chunks = per_w // CHUNK
    mesh = plsc.VectorSubcoreMesh(
        core_axis_name="c", subcore_axis_name="s", num_cores=NC, num_subcores=NS
    )

    @functools.partial(
        pl.kernel,
        out_type=(
            jax.ShapeDtypeStruct((n_total, N_HEADS), jnp.float32),
            jax.ShapeDtypeStruct((n_total // ROW, ROW), jnp.int32),
        ),
        mesh=mesh,
        scratch_types=[
            pltpu.VMEM((CHUNK,), jnp.float32),
            pltpu.VMEM((ROWS, ROW), jnp.int32),
            pltpu.VMEM((CHUNK, N_HEADS), jnp.float32),
            pltpu.SemaphoreType.DMA,
        ],
        compiler_params=pltpu.CompilerParams(use_tc_tiling_on_sc=False),
    )
    def body(coord_hbm, table_hbm, emb_hbm, idx_hbm, coord_v, idx_v, rows_v, sem):
        wid = lax.axis_index("s") * NC + lax.axis_index("c")
        wbase = wid * per_w
        scale = jnp.float32(N_POS / (MAX_POS - MIN_POS))

        def chunk_body(c, carry):
            base = pl.multiple_of(wbase + c * CHUNK, CHUNK)
            pltpu.sync_copy(coord_hbm.at[pl.ds(base, CHUNK)], coord_v)
            descs = []
            for j in range(ROWS):
                for u in range(ROW // L):
                    v = coord_v[pl.ds(j * ROW + u * L, L)]
                    pos = jnp.clip((v - MIN_POS) * scale, 0.0, float(N_POS))
                    idx_v[j, pl.ds(u * L, L)] = pos.astype(jnp.int32)
                descs.append(
                    pltpu.async_copy(
                        table_hbm.at[idx_v.at[j]],
                        rows_v.at[pl.ds(j * ROW, ROW)],
                        sem,
                    )
                )
            for d in descs:
                d.wait()
            pltpu.sync_copy(rows_v, emb_hbm.at[pl.ds(base, CHUNK)])
            pltpu.sync_copy(
                idx_v, idx_hbm.at[pl.ds(pl.multiple_of(base // ROW, ROWS), ROWS)]
            )
            return carry

        lax.fori_loop(0, chunks, chunk_body, 0, unroll=False)

    return body


def kernel(coord, embeddings_table):
    m, k = coord.shape
    n_total = m * k
    coord_flat = coord.reshape(n_total)
    emb_flat, idx_2d = _make_sc_kernel(n_total)(coord_flat, embeddings_table)
    return emb_flat.reshape(m, k, N_HEADS), idx_2d.reshape(m, k)


# speedup vs baseline: 11.0919x; 1.8135x over previous
import functools

import jax
import jax.numpy as jnp
from jax import lax
from jax.experimental import pallas as pl
from jax.experimental.pallas import tpu as pltpu
from jax.experimental.pallas import tpu_sc as plsc

MIN_POS = 0.0
MAX_POS = 1.0
N_POS = 100000
N_HEADS = 16

NC = 2
NS = 16
NW = NC * NS
L = 16

JB = 8
IB = 128
TH = N_HEADS // JB


def _make_sc_kernel(n_i, n_j):
    units = (n_j // JB) * (n_i // IB)
    per_w = units // NW
    ti_count = n_i // IB
    mesh = plsc.VectorSubcoreMesh(
        core_axis_name="c", subcore_axis_name="s", num_cores=NC, num_subcores=NS
    )

    @functools.partial(
        pl.kernel,
        out_type=(
            jax.ShapeDtypeStruct((n_j, TH, ti_count, JB, IB), jnp.float32),
            jax.ShapeDtypeStruct((n_j // JB, ti_count, JB, IB), jnp.int32),
        ),
        mesh=mesh,
        scratch_types=[
            pltpu.VMEM((JB, IB), jnp.float32),
            pltpu.VMEM((JB, IB), jnp.int32),
            pltpu.VMEM((JB, IB, N_HEADS), jnp.float32),
            pltpu.VMEM((JB, TH, JB, IB), jnp.float32),
            pltpu.SemaphoreType.DMA,
        ],
        compiler_params=pltpu.CompilerParams(
            use_tc_tiling_on_sc=False, needs_layout_passes=False
        ),
    )
    def body(coord_hbm, table_hbm, emb_hbm, idx_hbm, coord_v, idx_v, rows_v, out_v, sem):
        wid = lax.axis_index("s") * NC + lax.axis_index("c")
        scale = jnp.float32(N_POS / (MAX_POS - MIN_POS))
        lane = lax.iota(jnp.int32, L)

        def unit_body(u, carry):
            j8 = u // ti_count
            ti = u % ti_count
            j0 = pl.multiple_of(j8 * JB, JB)
            i0 = pl.multiple_of(ti * IB, IB)
            pltpu.sync_copy(coord_hbm.at[pl.ds(j0, JB), pl.ds(i0, IB)], coord_v)
            descs = []
            for jm in range(JB):
                for c in range(IB // L):
                    v = coord_v[jm, pl.ds(c * L, L)]
                    pos = jnp.clip((v - MIN_POS) * scale, 0.0, float(N_POS))
                    idx_v[jm, pl.ds(c * L, L)] = pos.astype(jnp.int32)
                descs.append(
                    pltpu.async_copy(
                        table_hbm.at[idx_v.at[jm]], rows_v.at[jm], sem
                    )
                )
            pltpu.sync_copy(idx_v, idx_hbm.at[j8, ti])
            for d in descs:
                d.wait()
            for jm in range(JB):
                for h in range(N_HEADS):
                    hvec = jnp.full((L,), h, jnp.int32)
                    jvec = jnp.full((L,), jm, jnp.int32)
                    for c in range(IB // L):
                        vec = plsc.load_gather(
                            rows_v, [jvec, lane + (c * L), hvec]
                        )
                        out_v[jm, h // JB, h % JB, pl.ds(c * L, L)] = vec
            for th in range(TH):
                pltpu.sync_copy(
                    out_v.at[:, th], emb_hbm.at[pl.ds(j0, JB), th, ti]
                )
            return carry

        lax.fori_loop(wid * per_w, (wid + 1) * per_w, unit_body, 0, unroll=False)

    return body


def kernel(coord, embeddings_table):
    n_i, n_j = coord.shape
    coord_t = coord.T
    emb5, idx4 = _make_sc_kernel(n_i, n_j)(coord_t, embeddings_table)
    emb = jnp.transpose(emb5, (2, 4, 0, 1, 3)).reshape(n_i, n_j, N_HEADS)
    idx = jnp.transpose(idx4, (1, 3, 0, 2)).reshape(n_i, n_j)
    return emb, idx
